# baseline (speedup 1.0000x reference)
import functools

import jax
import jax.numpy as jnp
from jax import lax
from jax.experimental import pallas as pl
from jax.experimental.pallas import tpu as pltpu
from jax.experimental.pallas import tpu_sc as plsc


_SEQ = 4096
_BATCH = 2
_D = 1024
_LANES = 16
_R = 8
_NBUF = 2


def _sc_body(x_hbm, pos_hbm, out_hbm,
             xb0, xb1, pb0, pb1, ob0, ob1,
             sx0, sx1, sp0, sp1, so0, so1):
    nc = 2
    rows_per_w = _SEQ // (nc * 16)
    steps = rows_per_w // _R
    wid = lax.axis_index("s") * nc + lax.axis_index("c")
    base = wid * rows_per_w

    xb = [xb0, xb1]
    pb = [pb0, pb1]
    ob = [ob0, ob1]
    sx = [sx0, sx1]
    sp = [sp0, sp1]
    so = [so0, so1]

    def start_in(step, slot):
        row = base + step * _R
        hx = pltpu.async_copy(x_hbm.at[pl.ds(row, _R)], xb[slot], sx[slot])
        hp = pltpu.async_copy(pos_hbm.at[pl.ds(row, _R)], pb[slot], sp[slot])
        return hx, hp

    def compute(slot):
        xref, pref, oref = xb[slot], pb[slot], ob[slot]

        def row_body(r, carry):
            for j in range(_D // _LANES):
                sl = pl.ds(j * _LANES, _LANES)
                p = pref[r, sl]
                oref[r, 0, sl] = xref[r, 0, sl] + p
                oref[r, 1, sl] = xref[r, 1, sl] + p
            return carry

        lax.fori_loop(0, _R, row_body, 0)

    in_h = {}
    out_h = {}
    for g in range(min(_NBUF, steps)):
        in_h[g] = start_in(g, g % _NBUF)
    for g in range(steps):
        slot = g % _NBUF
        hx, hp = in_h.pop(g)
        hx.wait()
        hp.wait()
        if g >= _NBUF:
            out_h.pop(g - _NBUF).wait()
        compute(slot)
        row = base + g * _R
        out_h[g] = pltpu.async_copy(ob[slot], out_hbm.at[pl.ds(row, _R)],
                                    so[slot])
        if g + _NBUF < steps:
            in_h[g + _NBUF] = start_in(g + _NBUF, slot)
    for h in out_h.values():
        h.wait()


def kernel(x, pos_emb):
    seq_len, batch, d_model = x.shape
    mesh = plsc.VectorSubcoreMesh(core_axis_name="c", subcore_axis_name="s")
    sc_fn = functools.partial(
        pl.kernel,
        mesh=mesh,
        out_type=jax.ShapeDtypeStruct((seq_len, batch, d_model), x.dtype),
        scratch_types=[
            pltpu.VMEM((_R, _BATCH, _D), jnp.float32),
            pltpu.VMEM((_R, _BATCH, _D), jnp.float32),
            pltpu.VMEM((_R, _D), jnp.float32),
            pltpu.VMEM((_R, _D), jnp.float32),
            pltpu.VMEM((_R, _BATCH, _D), jnp.float32),
            pltpu.VMEM((_R, _BATCH, _D), jnp.float32),
            pltpu.SemaphoreType.DMA,
            pltpu.SemaphoreType.DMA,
            pltpu.SemaphoreType.DMA,
            pltpu.SemaphoreType.DMA,
            pltpu.SemaphoreType.DMA,
            pltpu.SemaphoreType.DMA,
        ],
    )(_sc_body)
    return sc_fn(x, pos_emb[:seq_len])


# speedup vs baseline: 1.1164x; 1.1164x over previous
import functools

import jax
import jax.numpy as jnp
from jax import lax
from jax.experimental import pallas as pl
from jax.experimental.pallas import tpu as pltpu
from jax.experimental.pallas import tpu_sc as plsc


_SEQ = 4096
_BATCH = 2
_D = 1024
_LANES = 16
_R = 16
_NBUF = 2


def _sc_body(x_hbm, pos_hbm, out_hbm,
             xb0, xb1, pb0, pb1,
             sx0, sx1, sp0, sp1, so0, so1):
    nc = 2
    rows_per_w = _SEQ // (nc * 16)
    steps = rows_per_w // _R
    wid = lax.axis_index("s") * nc + lax.axis_index("c")
    base = wid * rows_per_w

    xb = [xb0, xb1]
    pb = [pb0, pb1]
    sx = [sx0, sx1]
    sp = [sp0, sp1]
    so = [so0, so1]

    def start_in(step, slot):
        row = base + step * _R
        hx = pltpu.async_copy(x_hbm.at[pl.ds(row, _R)], xb[slot], sx[slot])
        hp = pltpu.async_copy(pos_hbm.at[pl.ds(row, _R)], pb[slot], sp[slot])
        return hx, hp

    def compute(slot):
        xref, pref = xb[slot], pb[slot]

        def row_body(r, carry):
            for j in range(_D // _LANES):
                sl = pl.ds(j * _LANES, _LANES)
                p = pref[r, sl]
                plsc.addupdate(xref.at[r, 0, sl], p)
                plsc.addupdate(xref.at[r, 1, sl], p)
            return carry

        lax.fori_loop(0, _R, row_body, 0)

    in_h = {}
    out_h = {}
    for g in range(min(_NBUF, steps)):
        in_h[g] = start_in(g, g % _NBUF)
    for g in range(steps):
        slot = g % _NBUF
        hx, hp = in_h.pop(g)
        hx.wait()
        hp.wait()
        compute(slot)
        row = base + g * _R
        out_h[g] = pltpu.async_copy(xb[slot], out_hbm.at[pl.ds(row, _R)],
                                    so[slot])
        if g + _NBUF < steps:
            out_h.pop(g).wait()
            in_h[g + _NBUF] = start_in(g + _NBUF, slot)
    for h in out_h.values():
        h.wait()


def kernel(x, pos_emb):
    seq_len, batch, d_model = x.shape
    mesh = plsc.VectorSubcoreMesh(core_axis_name="c", subcore_axis_name="s")
    sc_fn = functools.partial(
        pl.kernel,
        mesh=mesh,
        out_type=jax.ShapeDtypeStruct((seq_len, batch, d_model), x.dtype),
        scratch_types=[
            pltpu.VMEM((_R, _BATCH, _D), jnp.float32),
            pltpu.VMEM((_R, _BATCH, _D), jnp.float32),
            pltpu.VMEM((_R, _D), jnp.float32),
            pltpu.VMEM((_R, _D), jnp.float32),
            pltpu.SemaphoreType.DMA,
            pltpu.SemaphoreType.DMA,
            pltpu.SemaphoreType.DMA,
            pltpu.SemaphoreType.DMA,
            pltpu.SemaphoreType.DMA,
            pltpu.SemaphoreType.DMA,
        ],
    )(_sc_body)
    return sc_fn(x, pos_emb[:seq_len])


# speedup vs baseline: 1.2123x; 1.0859x over previous
import functools

import jax
import jax.numpy as jnp
from jax import lax
from jax.experimental import pallas as pl
from jax.experimental.pallas import tpu as pltpu
from jax.experimental.pallas import tpu_sc as plsc


_SEQ = 4096
_BATCH = 2
_D = 1024
_LANES = 16
_R = 8
_NBUF = 4
_AHEAD = 2


def _sc_body(x_hbm, pos_hbm, out_hbm, *refs):
    xb = refs[0:_NBUF]
    pb = refs[_NBUF:2 * _NBUF]
    sx = refs[2 * _NBUF:3 * _NBUF]
    sp = refs[3 * _NBUF:4 * _NBUF]
    so = refs[4 * _NBUF:5 * _NBUF]

    nc = 2
    rows_per_w = _SEQ // (nc * 16)
    steps = rows_per_w // _R
    wid = lax.axis_index("s") * nc + lax.axis_index("c")
    base = wid * rows_per_w

    def start_in(step):
        slot = step % _NBUF
        row = base + step * _R
        hx = pltpu.async_copy(x_hbm.at[pl.ds(row, _R)], xb[slot], sx[slot])
        hp = pltpu.async_copy(pos_hbm.at[pl.ds(row, _R)], pb[slot], sp[slot])
        return hx, hp

    def compute(slot):
        xref, pref = xb[slot], pb[slot]

        def row_body(r, carry):
            for j in range(_D // _LANES):
                sl = pl.ds(j * _LANES, _LANES)
                p = pref[r, sl]
                plsc.addupdate(xref.at[r, 0, sl], p)
                plsc.addupdate(xref.at[r, 1, sl], p)
            return carry

        lax.fori_loop(0, _R, row_body, 0)

    in_h = {}
    out_h = {}
    for g in range(min(_AHEAD, steps)):
        in_h[g] = start_in(g)
    for g in range(steps):
        slot = g % _NBUF
        hx, hp = in_h.pop(g)
        hx.wait()
        hp.wait()
        compute(slot)
        out_h[g] = pltpu.async_copy(xb[slot], out_hbm.at[pl.ds(base + g * _R, _R)],
                                    so[slot])
        nxt = g + _AHEAD
        if nxt < steps:
            prev = nxt - _NBUF
            if prev >= 0:
                out_h.pop(prev).wait()
            in_h[nxt] = start_in(nxt)
    for h in out_h.values():
        h.wait()


def kernel(x, pos_emb):
    seq_len, batch, d_model = x.shape
    mesh = plsc.VectorSubcoreMesh(core_axis_name="c", subcore_axis_name="s")
    scratch = (
        [pltpu.VMEM((_R, _BATCH, _D), jnp.float32)] * _NBUF
        + [pltpu.VMEM((_R, _D), jnp.float32)] * _NBUF
        + [pltpu.SemaphoreType.DMA] * (3 * _NBUF)
    )
    sc_fn = functools.partial(
        pl.kernel,
        mesh=mesh,
        out_type=jax.ShapeDtypeStruct((seq_len, batch, d_model), x.dtype),
        scratch_types=scratch,
    )(_sc_body)
    return sc_fn(x, pos_emb[:seq_len])
